# baseline (speedup 1.0000x reference)
import functools

import jax
import jax.numpy as jnp
from jax import lax
from jax.experimental import pallas as pl
from jax.experimental.pallas import tpu as pltpu
from jax.experimental.pallas import tpu_sc as plsc

C = 128
N = 50000
NPAD = 50048
NBLK = NPAD // C
E = 600000
EPAD = 602112
ECH = 294
NB = 2
NG = ECH // NB
EL = 200000
ELPAD = 200704
LCH = 49
NSUB = 16
NCORE = 2
NEG = -3.0e38

_mesh = functools.partial(
    plsc.VectorSubcoreMesh, core_axis_name="c", subcore_axis_name="s",
    num_cores=NCORE, num_subcores=NSUB)


def _zero_vec(ref, n16):
    for i in range(n16):
        ref[pl.ds(i * 16, 16)] = jnp.zeros((16,), jnp.float32)


def _deg_kernel(i0, i1, i2, i3, d0, d1, d2, d3, ib, ones, zb, acc, sem, isem):
    core = lax.axis_index("c")
    sub = lax.axis_index("s")
    for i in range(8):
        ones[pl.ds(i * 16, 16)] = jnp.full((16,), 1.0, jnp.float32)
    _zero_vec(zb, 8)
    for a, (idx, out) in enumerate(((i0, d0), (i1, d1), (i2, d2), (i3, d3))):
        @pl.when(core == a // 2)
        def _():
            def zloop(j, _):
                ch = sub + j * NSUB

                @pl.when(ch < NBLK)
                def _():
                    pltpu.sync_copy(zb, acc.at[pl.ds(ch * C, C)])
                return 0
            lax.fori_loop(0, 25, zloop, 0)
            plsc.subcore_barrier()

            def eloop(g, _):
                s0 = lax.rem(g, 3)
                s1 = lax.rem(g + 1, 3)
                g8 = g * NB

                @pl.when(g == 0)
                def _():
                    pltpu.sync_copy(idx.at[sub, pl.ds(0, NB)], ib.at[0])

                @pl.when(g + 1 < NG)
                def _():
                    pltpu.async_copy(idx.at[sub, pl.ds(g8 + NB, NB)],
                                     ib.at[s1], isem)

                @pl.when(g >= 1)
                def _():
                    for b in range(NB):
                        pltpu.make_async_copy(ones, acc.at[ib.at[s0, b]],
                                              sem).wait()
                for b in range(NB):
                    pltpu.async_copy(ones, acc.at[ib.at[s0, b]], sem,
                                     add=True)

                @pl.when(g + 1 < NG)
                def _():
                    pltpu.make_async_copy(idx.at[sub, pl.ds(g8 + NB, NB)],
                                          ib.at[s1], isem).wait()
                return 0
            lax.fori_loop(0, NG, eloop, 0)
            for b in range(NB):
                pltpu.make_async_copy(ones, acc.at[ib.at[0, b]], sem).wait()
            plsc.subcore_barrier()

            def dloop(j, _):
                ch = sub + j * NSUB

                @pl.when(ch < NBLK)
                def _():
                    pltpu.sync_copy(acc.at[pl.ds(ch * C, C)],
                                    out.at[pl.ds(ch * C, C)])
                return 0
            lax.fori_loop(0, 25, dloop, 0)
            plsc.subcore_barrier()


def _degrees(s01, d01, s10, d10):
    k = pl.kernel(
        _deg_kernel,
        out_type=[jax.ShapeDtypeStruct((NPAD,), jnp.float32)] * 4,
        mesh=_mesh(),
        compiler_params=pltpu.CompilerParams(use_tc_tiling_on_sc=False),
        scratch_types=[
            pltpu.VMEM((3, NB, C), jnp.int32),
            pltpu.VMEM((C,), jnp.float32),
            pltpu.VMEM((C,), jnp.float32),
            pltpu.VMEM_SHARED((NPAD,), jnp.float32),
            pltpu.SemaphoreType.DMA,
            pltpu.SemaphoreType.DMA,
        ],
    )
    return k(s01, d01, s10, d10)


def _gather_rows_kernel(x, perm, out, pv, xv, sem):
    core = lax.axis_index("c")
    sub = lax.axis_index("s")

    @pl.when(jnp.logical_and(core == 0, sub == 0))
    def _():
        pltpu.sync_copy(perm, pv)
        pltpu.async_copy(x.at[pv], xv, sem).wait()
        pltpu.sync_copy(xv, out)


def _gather_rows(x_pad, perm):
    k = pl.kernel(
        _gather_rows_kernel,
        out_type=jax.ShapeDtypeStruct((C, C), jnp.float32),
        mesh=_mesh(),
        compiler_params=pltpu.CompilerParams(use_tc_tiling_on_sc=False),
        scratch_types=[
            pltpu.VMEM((C,), jnp.int32),
            pltpu.VMEM((C, C), jnp.float32),
            pltpu.SemaphoreType.DMA,
        ],
    )
    return k(x_pad, perm)


def _agg_kernel(h0, h1, h2, h3, src, dst, ddi, wrow,
                u0, u1, u2, u3, sib, dib, rb, vb, zb, dv, wv, acc,
                gsem, ssem, isem):
    core = lax.axis_index("c")
    sub = lax.axis_index("s")

    def zrow(r, _):
        zb[r, pl.ds(0, 16)] = jnp.zeros((16,), jnp.float32)
        zb[r, pl.ds(16, 16)] = jnp.zeros((16,), jnp.float32)
        return 0
    lax.fori_loop(0, C, zrow, 0)

    for k, (hk, uk) in enumerate(((h0, u0), (h1, u1), (h2, u2), (h3, u3))):
        @pl.when(core == k // 2)
        def _():
            def zloop(j, _):
                ch = sub + j * NSUB

                @pl.when(ch < NBLK)
                def _():
                    pltpu.sync_copy(zb, acc.at[pl.ds(ch * C, C)])
                return 0
            lax.fori_loop(0, 25, zloop, 0)
            pltpu.sync_copy(wrow.at[pl.ds(k * 32, 32)], wv)
            plsc.subcore_barrier()

            def eloop(g, _):
                p2 = lax.rem(g, 2)
                q2 = 1 - p2
                s0 = lax.rem(g, 3)
                s1 = lax.rem(g + 1, 3)
                g8 = g * NB

                @pl.when(g == 0)
                def _():
                    pltpu.sync_copy(src.at[sub, pl.ds(0, NB)], sib.at[0])
                    pltpu.sync_copy(dst.at[sub, pl.ds(0, NB)], dib.at[0])
                    for b in range(NB):
                        pltpu.async_copy(hk.at[sib.at[0, b]], rb.at[0, b],
                                         gsem)

                @pl.when(g + 1 < NG)
                def _():
                    pltpu.async_copy(src.at[sub, pl.ds(g8 + NB, NB)],
                                     sib.at[s1], isem)
                    pltpu.async_copy(dst.at[sub, pl.ds(g8 + NB, NB)],
                                     dib.at[s1], isem)
                for b in range(NB):
                    pltpu.make_async_copy(hk.at[sib.at[s0, b]],
                                          rb.at[p2, b], gsem).wait()
                for b in range(NB):
                    pltpu.async_copy(rb.at[p2, b], acc.at[dib.at[s0, b]],
                                     ssem, add=True)

                @pl.when(g + 1 < NG)
                def _():
                    pltpu.make_async_copy(src.at[sub, pl.ds(g8 + NB, NB)],
                                          sib.at[s1], isem).wait()
                    pltpu.make_async_copy(dst.at[sub, pl.ds(g8 + NB, NB)],
                                          dib.at[s1], isem).wait()

                    @pl.when(g >= 1)
                    def _():
                        for b in range(NB):
                            pltpu.make_async_copy(hk.at[sib.at[s1, b]],
                                                  rb.at[q2, b], ssem).wait()
                    for b in range(NB):
                        pltpu.async_copy(hk.at[sib.at[s1, b]], rb.at[q2, b],
                                         gsem)
                return 0
            lax.fori_loop(0, NG, eloop, 0)
            for b in range(2 * NB):
                pltpu.make_async_copy(hk.at[sib.at[0, 0]], rb.at[0, 0],
                                      ssem).wait()
            plsc.subcore_barrier()

            def dloop(j, _):
                ch = sub + j * NSUB

                @pl.when(ch < NBLK)
                def _():
                    r0 = ch * C
                    pltpu.sync_copy(acc.at[pl.ds(r0, C)], vb)
                    pltpu.sync_copy(ddi.at[pl.ds(r0, C)], dv)

                    def rbody(g, _):
                        sv = dv[pl.ds(g * 16, 16)]
                        for t in range(16):
                            r = g * 16 + t
                            sc = sv[t]
                            lo = jnp.maximum(vb[r, pl.ds(0, 16)], 0.0)
                            hi = jnp.maximum(vb[r, pl.ds(16, 16)], 0.0)
                            vb[r, pl.ds(0, 16)] = lo * wv[pl.ds(0, 16)] * sc
                            vb[r, pl.ds(16, 16)] = hi * wv[pl.ds(16, 16)] * sc
                        return 0
                    lax.fori_loop(0, 8, rbody, 0)
                    pltpu.sync_copy(vb, uk.at[pl.ds(r0, C)])
                return 0
            lax.fori_loop(0, 25, dloop, 0)
            plsc.subcore_barrier()


def _aggregate(h_slices, src, dst, ddi, wrow):
    k = pl.kernel(
        _agg_kernel,
        out_type=[jax.ShapeDtypeStruct((NPAD, 32), jnp.float32)] * 4,
        mesh=_mesh(),
        compiler_params=pltpu.CompilerParams(use_tc_tiling_on_sc=False),
        scratch_types=[
            pltpu.VMEM((3, NB, C), jnp.int32),
            pltpu.VMEM((3, NB, C), jnp.int32),
            pltpu.VMEM((2, NB, C, 32), jnp.float32),
            pltpu.VMEM((C, 32), jnp.float32),
            pltpu.VMEM((C, 32), jnp.float32),
            pltpu.VMEM((C,), jnp.float32),
            pltpu.VMEM((32,), jnp.float32),
            pltpu.VMEM_SHARED((NPAD, 32), jnp.float32),
            pltpu.SemaphoreType.DMA,
            pltpu.SemaphoreType.DMA,
            pltpu.SemaphoreType.DMA,
        ],
    )
    return k(*h_slices, src, dst, ddi, wrow)


def _link_kernel(l0, l1, u0, u1, u2, u3, v0, v1, v2, v3, bvec, out,
                 i0b, i1b, ub0, ub1, ub2, ub3, vb0, vb1, vb2, vb3, res, bsv,
                 sem, isem):
    core = lax.axis_index("c")
    sub = lax.axis_index("s")
    w = sub * NCORE + core
    us = (ub0, ub1, ub2, ub3)
    vs = (vb0, vb1, vb2, vb3)
    pltpu.sync_copy(bvec, bsv)

    def chunk(j, _):
        p = lax.rem(j, 2)
        q = 1 - p

        @pl.when(j == 0)
        def _():
            pltpu.sync_copy(l0.at[w, 0], i0b.at[0])
            pltpu.sync_copy(l1.at[w, 0], i1b.at[0])

        @pl.when(j + 1 < LCH)
        def _():
            pltpu.async_copy(l0.at[w, j + 1], i0b.at[q], isem)
            pltpu.async_copy(l1.at[w, j + 1], i1b.at[q], isem)
        gd = []
        for k, (uk, vk) in enumerate(((u0, v0), (u1, v1), (u2, v2), (u3, v3))):
            gd.append(pltpu.async_copy(uk.at[i0b.at[p]], us[k], sem))
            gd.append(pltpu.async_copy(vk.at[i1b.at[p]], vs[k], sem))
        for d in gd:
            d.wait()

        def edge(e, _):
            acc = us[0][e, pl.ds(0, 16)] * vs[0][e, pl.ds(0, 16)]
            acc = acc + us[0][e, pl.ds(16, 16)] * vs[0][e, pl.ds(16, 16)]
            for k in range(1, 4):
                acc = acc + us[k][e, pl.ds(0, 16)] * vs[k][e, pl.ds(0, 16)]
                acc = acc + us[k][e, pl.ds(16, 16)] * vs[k][e, pl.ds(16, 16)]
            res[e, pl.ds(0, 16)] = acc
            return 0
        lax.fori_loop(0, C, edge, 0)
        pltpu.sync_copy(res, out.at[w, j])

        @pl.when(j + 1 < LCH)
        def _():
            pltpu.make_async_copy(l0.at[w, j + 1], i0b.at[q], isem).wait()
            pltpu.make_async_copy(l1.at[w, j + 1], i1b.at[q], isem).wait()
        return 0
    lax.fori_loop(0, LCH, chunk, 0)


def _link(l0, l1, u_slices, v_slices, bvec):
    k = pl.kernel(
        _link_kernel,
        out_type=jax.ShapeDtypeStruct((NSUB * NCORE, LCH, C, 16), jnp.float32),
        mesh=_mesh(),
        compiler_params=pltpu.CompilerParams(use_tc_tiling_on_sc=False),
        scratch_types=[
            pltpu.VMEM((2, C), jnp.int32),
            pltpu.VMEM((2, C), jnp.int32),
        ] + [pltpu.VMEM((C, 32), jnp.float32)] * 8 + [
            pltpu.VMEM((C, 16), jnp.float32),
            pltpu.VMEM((16,), jnp.float32),
            pltpu.SemaphoreType.DMA,
            pltpu.SemaphoreType.DMA,
        ],
    )
    return k(l0, l1, *u_slices, *v_slices, bvec)


def _reduce_body(p, bsum, o):
    sel = (lax.broadcasted_iota(jnp.int32, (C, 8), 0) // 16
           == lax.broadcasted_iota(jnp.int32, (C, 8), 1))
    s = sel.astype(jnp.float32)
    mm = lax.dot_general(s, p[...], (((0,), (1,)), ((), ())),
                         preferred_element_type=jnp.float32)
    o[...] = (mm + bsum[0, 0]).reshape(1, 8, C)


def _reduce(partial2d, bsum):
    nrow = ELPAD * 16 // C
    return pl.pallas_call(
        _reduce_body,
        grid=(nrow // C,),
        in_specs=[pl.BlockSpec((C, C), lambda i: (i, 0)),
                  pl.BlockSpec((1, 16), lambda i: (0, 0))],
        out_specs=pl.BlockSpec((1, 8, C), lambda i: (i, 0, 0)),
        out_shape=jax.ShapeDtypeStruct((nrow // C, 8, C), jnp.float32),
    )(partial2d, bsum)


BR = 2944
NGRID = NPAD // BR


def _dense1_body(xu, xi, pu, pi, da, db, dc, dd, su, si, oa, ob, oc, od):
    i = pl.program_id(0)
    ridx = i * BR + lax.broadcasted_iota(jnp.int32, (BR, 1), 0)
    keep = ridx < N

    def one(x, p, out):
        nrm = jnp.sqrt(jnp.sum(p[...] * p[...])) + 1e-16
        s = jnp.dot(x[...], p[...], preferred_element_type=jnp.float32) / nrm
        out[...] = jnp.where(keep, s, NEG)
    one(xu, pu, su)
    one(xi, pi, si)
    for x, o in ((da, oa), (db, ob), (dc, oc), (dd, od)):
        v = x[...]
        o[...] = jnp.where(v > 0.0, lax.rsqrt(v), 0.0)


def _dense1(xu_pad, xi_pad, pu, pi, degs):
    col = pl.BlockSpec((BR, 1), lambda i: (i, 0))
    return pl.pallas_call(
        _dense1_body,
        grid=(NGRID,),
        in_specs=[
            pl.BlockSpec((BR, C), lambda i: (i, 0)),
            pl.BlockSpec((BR, C), lambda i: (i, 0)),
            pl.BlockSpec((C, 1), lambda i: (0, 0)),
            pl.BlockSpec((C, 1), lambda i: (0, 0)),
            col, col, col, col,
        ],
        out_specs=[col] * 6,
        out_shape=[jax.ShapeDtypeStruct((NPAD, 1), jnp.float32)] * 6,
    )(xu_pad, xi_pad, pu, pi, *degs)


def _topk_body(sref, tv_ref, pm_ref, s):
    def cp(r, _):
        s[pl.ds(r, 1), :] = sref[pl.ds(r, 1), :]
        return 0
    lax.fori_loop(0, NBLK, cp, 0)
    flat = (lax.broadcasted_iota(jnp.int32, (NBLK, C), 0) * C
            + lax.broadcasted_iota(jnp.int32, (NBLK, C), 1))
    rows = lax.broadcasted_iota(jnp.int32, (C, 1), 0)
    lanes = lax.broadcasted_iota(jnp.int32, (1, C), 1)

    def step(i, carry):
        tv, pm = carry
        arr = s[...]
        m = jnp.max(arr)
        fi = jnp.min(jnp.where(arr == m, flat, jnp.int32(2 ** 30)))
        tv = jnp.where(rows == i, m, tv)
        pm = jnp.where(rows == i, fi, pm)
        row = fi // C
        col = fi - row * C
        old = s[pl.ds(row, 1), :]
        s[pl.ds(row, 1), :] = jnp.where(lanes == col, NEG, old)
        return tv, pm

    tv0 = jnp.zeros((C, 1), jnp.float32)
    pm0 = jnp.zeros((C, 1), jnp.int32)
    tv, pm = lax.fori_loop(0, C, step, (tv0, pm0))
    tv_ref[...] = tv
    pm_ref[...] = pm


def _topk(score):
    return pl.pallas_call(
        _topk_body,
        in_specs=[pl.BlockSpec((NBLK, C), lambda: (0, 0))],
        out_specs=[pl.BlockSpec((C, 1), lambda: (0, 0))] * 2,
        out_shape=[jax.ShapeDtypeStruct((C, 1), jnp.float32),
                   jax.ShapeDtypeStruct((C, 1), jnp.int32)],
        scratch_shapes=[pltpu.VMEM((NBLK, C), jnp.float32)],
    )(score)


def _gru_body(xt, tv, wih, whh, bih, bhh, w0, wp, bp, w_out, wrow, bsum):
    x = xt[...] * jnp.tanh(tv[...])
    gi = lax.dot_general(x, wih[...], (((1,), (1,)), ((), ())),
                         preferred_element_type=jnp.float32) + bih[...]
    gh = lax.dot_general(w0[...], whh[...], (((1,), (1,)), ((), ())),
                         preferred_element_type=jnp.float32) + bhh[...]
    r = jax.nn.sigmoid(gi[:, 0:C] + gh[:, 0:C])
    z = jax.nn.sigmoid(gi[:, C:2 * C] + gh[:, C:2 * C])
    n = jnp.tanh(gi[:, 2 * C:3 * C] + r * gh[:, 2 * C:3 * C])
    w_out[...] = (1.0 - z) * n + z * w0[...]
    wrow[...] = wp[0:1, :] + wp[1:2, :]
    bsum[...] = jnp.zeros((1, 16), jnp.float32) + (bp[0, 0] + bp[0, 1])


def _gru(xt, tv, wih, whh, bih, bhh, w0, wp, bp):
    full = lambda s: pl.BlockSpec(s, lambda: tuple(0 for _ in s))
    return pl.pallas_call(
        _gru_body,
        in_specs=[full((C, C)), full((C, 1)), full((3 * C, C)),
                  full((3 * C, C)), full((1, 3 * C)), full((1, 3 * C)),
                  full((C, C)), full((2, C)), full((1, 2))],
        out_specs=[full((C, C)), full((1, C)), full((1, 16))],
        out_shape=[jax.ShapeDtypeStruct((C, C), jnp.float32),
                   jax.ShapeDtypeStruct((1, C), jnp.float32),
                   jax.ShapeDtypeStruct((1, 16), jnp.float32)],
    )(xt, tv, wih, whh, bih, bhh, w0, wp, bp)


def _htab_body(x, dsi, w, o0, o1, o2, o3):
    h = jnp.dot(x[...] * dsi[...], w[...],
                preferred_element_type=jnp.float32)
    for k, o in enumerate((o0, o1, o2, o3)):
        o[...] = h[:, k * 32:(k + 1) * 32]


def _htables(x_pad, dsi, w):
    return pl.pallas_call(
        _htab_body,
        grid=(NGRID,),
        in_specs=[
            pl.BlockSpec((BR, C), lambda i: (i, 0)),
            pl.BlockSpec((BR, 1), lambda i: (i, 0)),
            pl.BlockSpec((C, C), lambda i: (0, 0)),
        ],
        out_specs=[pl.BlockSpec((BR, 32), lambda i: (i, 0))] * 4,
        out_shape=[jax.ShapeDtypeStruct((NPAD, 32), jnp.float32)] * 4,
    )(x_pad, dsi, w)


def _pad_edges(ei):
    pad = N + (jnp.arange(EPAD - E, dtype=jnp.int32) % (NPAD - N))
    src = jnp.concatenate([ei[0].astype(jnp.int32), pad]).reshape(NSUB, ECH, C)
    dst = jnp.concatenate([ei[1].astype(jnp.int32), pad]).reshape(NSUB, ECH, C)
    return src, dst


def _pad_rows(x):
    return jnp.concatenate([x, jnp.zeros((NPAD - N, C), x.dtype)], axis=0)


def kernel(x_user, x_item, edge_index_u2i, edge_index_i2u, edge_label_index,
           p_u2i, Wih_u2i, Whh_u2i, bih_u2i, bhh_u2i, W0_u2i,
           p_i2u, Wih_i2u, Whh_i2u, bih_i2u, bhh_i2u, W0_i2u, W_post, b_post):
    xu = _pad_rows(x_user)
    xi = _pad_rows(x_item)
    s01, d01 = _pad_edges(edge_index_u2i)
    s10, d10 = _pad_edges(edge_index_i2u)

    dg_s01, dg_d01, dg_s10, dg_d10 = _degrees(s01, d01, s10, d10)
    su, si, dsi_u2i, ddi_u2i, dsi_i2u, ddi_i2u = _dense1(
        xu, xi, p_u2i.reshape(C, 1), p_i2u.reshape(C, 1),
        [d.reshape(NPAD, 1) for d in (dg_s01, dg_d01, dg_s10, dg_d10)])
    tv_u, pm_u = _topk(su.reshape(NBLK, C))
    tv_i, pm_i = _topk(si.reshape(NBLK, C))
    xt_u = _gather_rows(xu, pm_u.reshape(C))
    xt_i = _gather_rows(xi, pm_i.reshape(C))
    w_u2i, wrow, bsum = _gru(xt_u, tv_u, Wih_u2i, Whh_u2i,
                             bih_u2i.reshape(1, 3 * C),
                             bhh_u2i.reshape(1, 3 * C), W0_u2i,
                             W_post, b_post.reshape(1, 2))
    w_i2u, _, _ = _gru(xt_i, tv_i, Wih_i2u, Whh_i2u,
                       bih_i2u.reshape(1, 3 * C),
                       bhh_i2u.reshape(1, 3 * C), W0_i2u,
                       W_post, b_post.reshape(1, 2))

    h_u2i = _htables(xu, dsi_u2i, w_u2i)
    h_i2u = _htables(xi, dsi_i2u, w_i2u)
    ones = jnp.ones((C,), jnp.float32)
    v_sl = _aggregate(h_u2i, s01, d01, ddi_u2i.reshape(NPAD), ones)
    u_sl = _aggregate(h_i2u, s10, d10, ddi_i2u.reshape(NPAD),
                      wrow.reshape(C))

    lpad = jnp.zeros((ELPAD - EL,), jnp.int32)
    l0 = jnp.concatenate([edge_label_index[0].astype(jnp.int32), lpad])
    l1 = jnp.concatenate([edge_label_index[1].astype(jnp.int32), lpad])
    l0 = l0.reshape(NSUB * NCORE, LCH, C)
    l1 = l1.reshape(NSUB * NCORE, LCH, C)
    partial = _link(l0, l1, u_sl, v_sl, bsum.reshape(16))
    red = _reduce(partial.reshape(ELPAD * 16 // C, C), bsum)
    return jnp.transpose(red, (0, 2, 1)).reshape(ELPAD)[:EL]


# speedup vs baseline: 1.0952x; 1.0952x over previous
import functools

import jax
import jax.numpy as jnp
from jax import lax
from jax.experimental import pallas as pl
from jax.experimental.pallas import tpu as pltpu
from jax.experimental.pallas import tpu_sc as plsc

C = 128
N = 50000
NPAD = 50048
NBLK = NPAD // C
E = 600000
EPAD = 606208
ECH = 296
NB = 4
NG = ECH // NB
EL = 200000
ELPAD = 200704
LCH = 49
NSUB = 16
NCORE = 2
NEG = -3.0e38

_mesh = functools.partial(
    plsc.VectorSubcoreMesh, core_axis_name="c", subcore_axis_name="s",
    num_cores=NCORE, num_subcores=NSUB)


def _zero_vec(ref, n16):
    for i in range(n16):
        ref[pl.ds(i * 16, 16)] = jnp.zeros((16,), jnp.float32)


def _deg_kernel(i0, i1, i2, i3, d0, d1, d2, d3, ib, ones, zb, acc, sem, isem):
    core = lax.axis_index("c")
    sub = lax.axis_index("s")
    for i in range(8):
        ones[pl.ds(i * 16, 16)] = jnp.full((16,), 1.0, jnp.float32)
    _zero_vec(zb, 8)
    for a, (idx, out) in enumerate(((i0, d0), (i1, d1), (i2, d2), (i3, d3))):
        @pl.when(core == a // 2)
        def _():
            def zloop(j, _):
                ch = sub + j * NSUB

                @pl.when(ch < NBLK)
                def _():
                    pltpu.sync_copy(zb, acc.at[pl.ds(ch * C, C)])
                return 0
            lax.fori_loop(0, 25, zloop, 0)
            plsc.subcore_barrier()

            def eloop(g, _):
                p = lax.rem(g, 2)
                q = 1 - p
                g8 = g * NB

                @pl.when(g == 0)
                def _():
                    pltpu.sync_copy(idx.at[sub, pl.ds(0, NB)], ib.at[0])

                @pl.when(g + 1 < NG)
                def _():
                    pltpu.async_copy(idx.at[sub, pl.ds(g8 + NB, NB)],
                                     ib.at[q], isem)
                ad = [pltpu.async_copy(ones, acc.at[ib.at[p, b]], sem,
                                       add=True) for b in range(NB)]
                for d in ad:
                    d.wait()

                @pl.when(g + 1 < NG)
                def _():
                    pltpu.make_async_copy(idx.at[sub, pl.ds(g8 + NB, NB)],
                                          ib.at[q], isem).wait()
                return 0
            lax.fori_loop(0, NG, eloop, 0)
            plsc.subcore_barrier()

            def dloop(j, _):
                ch = sub + j * NSUB

                @pl.when(ch < NBLK)
                def _():
                    pltpu.sync_copy(acc.at[pl.ds(ch * C, C)], out.at[ch])
                return 0
            lax.fori_loop(0, 25, dloop, 0)
            plsc.subcore_barrier()


def _degrees(s01, d01, s10, d10):
    k = pl.kernel(
        _deg_kernel,
        out_type=[jax.ShapeDtypeStruct((NBLK, C), jnp.float32)] * 4,
        mesh=_mesh(),
        compiler_params=pltpu.CompilerParams(use_tc_tiling_on_sc=False),
        scratch_types=[
            pltpu.VMEM((2, NB, C), jnp.int32),
            pltpu.VMEM((C,), jnp.float32),
            pltpu.VMEM((C,), jnp.float32),
            pltpu.VMEM_SHARED((NPAD,), jnp.float32),
            pltpu.SemaphoreType.DMA,
            pltpu.SemaphoreType.DMA,
        ],
    )
    return k(s01, d01, s10, d10)


def _gather_rows_kernel(x, perm, out, pv, xv, sem):
    core = lax.axis_index("c")
    sub = lax.axis_index("s")

    @pl.when(jnp.logical_and(core == 0, sub == 0))
    def _():
        pltpu.sync_copy(perm, pv)
        pltpu.async_copy(x.at[pv], xv, sem).wait()
        pltpu.sync_copy(xv, out)


def _gather_rows(x_pad, perm):
    k = pl.kernel(
        _gather_rows_kernel,
        out_type=jax.ShapeDtypeStruct((C, C), jnp.float32),
        mesh=_mesh(),
        compiler_params=pltpu.CompilerParams(use_tc_tiling_on_sc=False),
        scratch_types=[
            pltpu.VMEM((C,), jnp.int32),
            pltpu.VMEM((C, C), jnp.float32),
            pltpu.SemaphoreType.DMA,
        ],
    )
    return k(x_pad, perm)


def _agg_kernel(h0, h1, h2, h3, src, dst, ddi, wrow,
                u0, u1, u2, u3, sib, dib, rb, vb, zb, dv, wv, acc,
                gsem, ssem, isem):
    core = lax.axis_index("c")
    sub = lax.axis_index("s")

    def zrow(r, _):
        zb[r, pl.ds(0, 16)] = jnp.zeros((16,), jnp.float32)
        zb[r, pl.ds(16, 16)] = jnp.zeros((16,), jnp.float32)
        return 0
    lax.fori_loop(0, C, zrow, 0)

    for k, (hk, uk) in enumerate(((h0, u0), (h1, u1), (h2, u2), (h3, u3))):
        @pl.when(core == k // 2)
        def _():
            def zloop(j, _):
                ch = sub + j * NSUB

                @pl.when(ch < NBLK)
                def _():
                    pltpu.sync_copy(zb, acc.at[pl.ds(ch * C, C)])
                return 0
            lax.fori_loop(0, 25, zloop, 0)
            pltpu.sync_copy(wrow.at[pl.ds(k * 32, 32)], wv)
            plsc.subcore_barrier()

            def eloop(g, _):
                p = lax.rem(g, 2)
                q = 1 - p
                g8 = g * NB

                @pl.when(g == 0)
                def _():
                    pltpu.sync_copy(src.at[sub, pl.ds(0, NB)], sib.at[0])
                    pltpu.sync_copy(dst.at[sub, pl.ds(0, NB)], dib.at[0])

                @pl.when(g + 1 < NG)
                def _():
                    pltpu.async_copy(src.at[sub, pl.ds(g8 + NB, NB)],
                                     sib.at[q], isem)
                    pltpu.async_copy(dst.at[sub, pl.ds(g8 + NB, NB)],
                                     dib.at[q], isem)
                gd = [pltpu.async_copy(hk.at[sib.at[p, b]], rb.at[b], gsem)
                      for b in range(NB)]
                for d in gd:
                    d.wait()
                sd = [pltpu.async_copy(rb.at[b], acc.at[dib.at[p, b]],
                                       ssem, add=True) for b in range(NB)]
                for d in sd:
                    d.wait()

                @pl.when(g + 1 < NG)
                def _():
                    pltpu.make_async_copy(src.at[sub, pl.ds(g8 + NB, NB)],
                                          sib.at[q], isem).wait()
                    pltpu.make_async_copy(dst.at[sub, pl.ds(g8 + NB, NB)],
                                          dib.at[q], isem).wait()
                return 0
            lax.fori_loop(0, NG, eloop, 0)
            plsc.subcore_barrier()

            def dloop(j, _):
                ch = sub + j * NSUB

                @pl.when(ch < NBLK)
                def _():
                    r0 = ch * C
                    pltpu.sync_copy(acc.at[pl.ds(r0, C)], vb)
                    pltpu.sync_copy(ddi.at[pl.ds(r0, C)], dv)

                    def rbody(g, _):
                        sv = dv[pl.ds(g * 16, 16)]
                        for t in range(16):
                            r = g * 16 + t
                            sc = sv[t]
                            lo = jnp.maximum(vb[r, pl.ds(0, 16)], 0.0)
                            hi = jnp.maximum(vb[r, pl.ds(16, 16)], 0.0)
                            vb[r, pl.ds(0, 16)] = lo * wv[pl.ds(0, 16)] * sc
                            vb[r, pl.ds(16, 16)] = hi * wv[pl.ds(16, 16)] * sc
                        return 0
                    lax.fori_loop(0, 8, rbody, 0)
                    pltpu.sync_copy(vb, uk.at[pl.ds(r0, C)])
                return 0
            lax.fori_loop(0, 25, dloop, 0)
            plsc.subcore_barrier()


def _aggregate(h_slices, src, dst, ddi, wrow):
    k = pl.kernel(
        _agg_kernel,
        out_type=[jax.ShapeDtypeStruct((NPAD, 32), jnp.float32)] * 4,
        mesh=_mesh(),
        compiler_params=pltpu.CompilerParams(use_tc_tiling_on_sc=False),
        scratch_types=[
            pltpu.VMEM((2, NB, C), jnp.int32),
            pltpu.VMEM((2, NB, C), jnp.int32),
            pltpu.VMEM((NB, C, 32), jnp.float32),
            pltpu.VMEM((C, 32), jnp.float32),
            pltpu.VMEM((C, 32), jnp.float32),
            pltpu.VMEM((C,), jnp.float32),
            pltpu.VMEM((32,), jnp.float32),
            pltpu.VMEM_SHARED((NPAD, 32), jnp.float32),
            pltpu.SemaphoreType.DMA,
            pltpu.SemaphoreType.DMA,
            pltpu.SemaphoreType.DMA,
        ],
    )
    return k(*h_slices, src, dst, ddi, wrow)


def _link_kernel(l0, l1, u0, u1, u2, u3, v0, v1, v2, v3, bvec, out,
                 i0b, i1b, ub0, ub1, ub2, ub3, vb0, vb1, vb2, vb3, res, bsv,
                 sem, isem):
    core = lax.axis_index("c")
    sub = lax.axis_index("s")
    w = sub * NCORE + core
    us = (ub0, ub1, ub2, ub3)
    vs = (vb0, vb1, vb2, vb3)
    pltpu.sync_copy(bvec, bsv)

    def chunk(j, _):
        p = lax.rem(j, 2)
        q = 1 - p

        @pl.when(j == 0)
        def _():
            pltpu.sync_copy(l0.at[w, 0], i0b.at[0])
            pltpu.sync_copy(l1.at[w, 0], i1b.at[0])

        @pl.when(j + 1 < LCH)
        def _():
            pltpu.async_copy(l0.at[w, j + 1], i0b.at[q], isem)
            pltpu.async_copy(l1.at[w, j + 1], i1b.at[q], isem)
        gd = []
        for k, (uk, vk) in enumerate(((u0, v0), (u1, v1), (u2, v2), (u3, v3))):
            gd.append(pltpu.async_copy(uk.at[i0b.at[p]], us[k], sem))
            gd.append(pltpu.async_copy(vk.at[i1b.at[p]], vs[k], sem))
        for d in gd:
            d.wait()

        def edge(e, _):
            acc = us[0][e, pl.ds(0, 16)] * vs[0][e, pl.ds(0, 16)]
            acc = acc + us[0][e, pl.ds(16, 16)] * vs[0][e, pl.ds(16, 16)]
            for k in range(1, 4):
                acc = acc + us[k][e, pl.ds(0, 16)] * vs[k][e, pl.ds(0, 16)]
                acc = acc + us[k][e, pl.ds(16, 16)] * vs[k][e, pl.ds(16, 16)]
            res[e, pl.ds(0, 16)] = acc
            return 0
        lax.fori_loop(0, C, edge, 0)
        pltpu.sync_copy(res, out.at[w, j])

        @pl.when(j + 1 < LCH)
        def _():
            pltpu.make_async_copy(l0.at[w, j + 1], i0b.at[q], isem).wait()
            pltpu.make_async_copy(l1.at[w, j + 1], i1b.at[q], isem).wait()
        return 0
    lax.fori_loop(0, LCH, chunk, 0)


def _link(l0, l1, u_slices, v_slices, bvec):
    k = pl.kernel(
        _link_kernel,
        out_type=jax.ShapeDtypeStruct((NSUB * NCORE, LCH, C, 16), jnp.float32),
        mesh=_mesh(),
        compiler_params=pltpu.CompilerParams(use_tc_tiling_on_sc=False),
        scratch_types=[
            pltpu.VMEM((2, C), jnp.int32),
            pltpu.VMEM((2, C), jnp.int32),
        ] + [pltpu.VMEM((C, 32), jnp.float32)] * 8 + [
            pltpu.VMEM((C, 16), jnp.float32),
            pltpu.VMEM((16,), jnp.float32),
            pltpu.SemaphoreType.DMA,
            pltpu.SemaphoreType.DMA,
        ],
    )
    return k(l0, l1, *u_slices, *v_slices, bvec)


def _reduce_body(p, bsum, o):
    sel = (lax.broadcasted_iota(jnp.int32, (C, 8), 0) // 16
           == lax.broadcasted_iota(jnp.int32, (C, 8), 1))
    s = sel.astype(jnp.float32)
    mm = lax.dot_general(s, p[...], (((0,), (1,)), ((), ())),
                         preferred_element_type=jnp.float32)
    o[...] = (mm + bsum[0, 0]).reshape(1, 8, C)


def _reduce(partial2d, bsum):
    nrow = ELPAD * 16 // C
    return pl.pallas_call(
        _reduce_body,
        grid=(nrow // C,),
        in_specs=[pl.BlockSpec((C, C), lambda i: (i, 0)),
                  pl.BlockSpec((1, 16), lambda i: (0, 0))],
        out_specs=pl.BlockSpec((1, 8, C), lambda i: (i, 0, 0)),
        out_shape=jax.ShapeDtypeStruct((nrow // C, 8, C), jnp.float32),
    )(partial2d, bsum)


BR = 2944
NGRID = NPAD // BR


def _dense1_body(xu, xi, pu, pi, da, dc, su, si, oa, oc):
    i = pl.program_id(0)
    ridx = i * BR + lax.broadcasted_iota(jnp.int32, (BR, 1), 0)
    keep = ridx < N

    def one(x, p, out):
        nrm = jnp.sqrt(jnp.sum(p[...] * p[...])) + 1e-16
        s = jnp.dot(x[...], p[...], preferred_element_type=jnp.float32) / nrm
        out[...] = jnp.where(keep, s, NEG)
    one(xu, pu, su)
    one(xi, pi, si)
    for x, o in ((da, oa), (dc, oc)):
        v = x[...]
        o[...] = jnp.where(v > 0.0, lax.rsqrt(v), 0.0)


def _dense1(xu_pad, xi_pad, pu, pi, degs):
    col = pl.BlockSpec((BR, 1), lambda i: (i, 0))
    return pl.pallas_call(
        _dense1_body,
        grid=(NGRID,),
        in_specs=[
            pl.BlockSpec((BR, C), lambda i: (i, 0)),
            pl.BlockSpec((BR, C), lambda i: (i, 0)),
            pl.BlockSpec((C, 1), lambda i: (0, 0)),
            pl.BlockSpec((C, 1), lambda i: (0, 0)),
            col, col,
        ],
        out_specs=[col] * 4,
        out_shape=[jax.ShapeDtypeStruct((NPAD, 1), jnp.float32)] * 4,
    )(xu_pad, xi_pad, pu, pi, *degs)


def _ddi_body(da, db, oa, ob):
    for x, o in ((da, oa), (db, ob)):
        v = x[...]
        o[...] = jnp.where(v > 0.0, lax.rsqrt(v), 0.0)


def _ddi(da, db):
    full = pl.BlockSpec((NBLK, C), lambda: (0, 0))
    return pl.pallas_call(
        _ddi_body,
        in_specs=[full, full],
        out_specs=[full, full],
        out_shape=[jax.ShapeDtypeStruct((NBLK, C), jnp.float32)] * 2,
    )(da, db)


def _topk_body(sref, tv_ref, pm_ref, s):
    def cp(r, _):
        s[pl.ds(r, 1), :] = sref[pl.ds(r, 1), :]
        return 0
    lax.fori_loop(0, NBLK, cp, 0)
    flat = (lax.broadcasted_iota(jnp.int32, (NBLK, C), 0) * C
            + lax.broadcasted_iota(jnp.int32, (NBLK, C), 1))
    rows = lax.broadcasted_iota(jnp.int32, (C, 1), 0)
    lanes = lax.broadcasted_iota(jnp.int32, (1, C), 1)

    def step(i, carry):
        tv, pm = carry
        arr = s[...]
        m = jnp.max(arr)
        fi = jnp.min(jnp.where(arr == m, flat, jnp.int32(2 ** 30)))
        tv = jnp.where(rows == i, m, tv)
        pm = jnp.where(rows == i, fi, pm)
        row = fi // C
        col = fi - row * C
        old = s[pl.ds(row, 1), :]
        s[pl.ds(row, 1), :] = jnp.where(lanes == col, NEG, old)
        return tv, pm

    tv0 = jnp.zeros((C, 1), jnp.float32)
    pm0 = jnp.zeros((C, 1), jnp.int32)
    tv, pm = lax.fori_loop(0, C, step, (tv0, pm0))
    tv_ref[...] = tv
    pm_ref[...] = pm


def _topk(score):
    return pl.pallas_call(
        _topk_body,
        in_specs=[pl.BlockSpec((NBLK, C), lambda: (0, 0))],
        out_specs=[pl.BlockSpec((C, 1), lambda: (0, 0))] * 2,
        out_shape=[jax.ShapeDtypeStruct((C, 1), jnp.float32),
                   jax.ShapeDtypeStruct((C, 1), jnp.int32)],
        scratch_shapes=[pltpu.VMEM((NBLK, C), jnp.float32)],
    )(score)


def _gru_body(xt, tv, wih, whh, bih, bhh, w0, wp, bp, w_out, wrow, bsum):
    x = xt[...] * jnp.tanh(tv[...])
    gi = lax.dot_general(x, wih[...], (((1,), (1,)), ((), ())),
                         preferred_element_type=jnp.float32) + bih[...]
    gh = lax.dot_general(w0[...], whh[...], (((1,), (1,)), ((), ())),
                         preferred_element_type=jnp.float32) + bhh[...]
    r = jax.nn.sigmoid(gi[:, 0:C] + gh[:, 0:C])
    z = jax.nn.sigmoid(gi[:, C:2 * C] + gh[:, C:2 * C])
    n = jnp.tanh(gi[:, 2 * C:3 * C] + r * gh[:, 2 * C:3 * C])
    w_out[...] = (1.0 - z) * n + z * w0[...]
    wrow[...] = wp[0:1, :] + wp[1:2, :]
    bsum[...] = jnp.zeros((1, 16), jnp.float32) + (bp[0, 0] + bp[0, 1])


def _gru(xt, tv, wih, whh, bih, bhh, w0, wp, bp):
    full = lambda s: pl.BlockSpec(s, lambda: tuple(0 for _ in s))
    return pl.pallas_call(
        _gru_body,
        in_specs=[full((C, C)), full((C, 1)), full((3 * C, C)),
                  full((3 * C, C)), full((1, 3 * C)), full((1, 3 * C)),
                  full((C, C)), full((2, C)), full((1, 2))],
        out_specs=[full((C, C)), full((1, C)), full((1, 16))],
        out_shape=[jax.ShapeDtypeStruct((C, C), jnp.float32),
                   jax.ShapeDtypeStruct((1, C), jnp.float32),
                   jax.ShapeDtypeStruct((1, 16), jnp.float32)],
    )(xt, tv, wih, whh, bih, bhh, w0, wp, bp)


def _htab_body(x, dsi, w, o0, o1, o2, o3):
    h = jnp.dot(x[...] * dsi[...], w[...],
                preferred_element_type=jnp.float32)
    for k, o in enumerate((o0, o1, o2, o3)):
        o[...] = h[:, k * 32:(k + 1) * 32]


def _htables(x_pad, dsi, w):
    return pl.pallas_call(
        _htab_body,
        grid=(NGRID,),
        in_specs=[
            pl.BlockSpec((BR, C), lambda i: (i, 0)),
            pl.BlockSpec((BR, 1), lambda i: (i, 0)),
            pl.BlockSpec((C, C), lambda i: (0, 0)),
        ],
        out_specs=[pl.BlockSpec((BR, 32), lambda i: (i, 0))] * 4,
        out_shape=[jax.ShapeDtypeStruct((NPAD, 32), jnp.float32)] * 4,
    )(x_pad, dsi, w)


def _pad_edges(ei):
    pad = N + (jnp.arange(EPAD - E, dtype=jnp.int32) % (NPAD - N))
    src = jnp.concatenate([ei[0].astype(jnp.int32), pad]).reshape(NSUB, ECH, C)
    dst = jnp.concatenate([ei[1].astype(jnp.int32), pad]).reshape(NSUB, ECH, C)
    return src, dst


def _pad_rows(x):
    return jnp.concatenate([x, jnp.zeros((NPAD - N, C), x.dtype)], axis=0)


def kernel(x_user, x_item, edge_index_u2i, edge_index_i2u, edge_label_index,
           p_u2i, Wih_u2i, Whh_u2i, bih_u2i, bhh_u2i, W0_u2i,
           p_i2u, Wih_i2u, Whh_i2u, bih_i2u, bhh_i2u, W0_i2u, W_post, b_post):
    xu = _pad_rows(x_user)
    xi = _pad_rows(x_item)
    s01, d01 = _pad_edges(edge_index_u2i)
    s10, d10 = _pad_edges(edge_index_i2u)

    dg_s01, dg_d01, dg_s10, dg_d10 = _degrees(s01, d01, s10, d10)
    su, si, dsi_u2i, dsi_i2u = _dense1(
        xu, xi, p_u2i.reshape(C, 1), p_i2u.reshape(C, 1),
        [d.reshape(NPAD, 1) for d in (dg_s01, dg_s10)])
    ddi_u2i, ddi_i2u = _ddi(dg_d01, dg_d10)
    tv_u, pm_u = _topk(su.reshape(NBLK, C))
    tv_i, pm_i = _topk(si.reshape(NBLK, C))
    xt_u = _gather_rows(xu, pm_u.reshape(C))
    xt_i = _gather_rows(xi, pm_i.reshape(C))
    w_u2i, wrow, bsum = _gru(xt_u, tv_u, Wih_u2i, Whh_u2i,
                             bih_u2i.reshape(1, 3 * C),
                             bhh_u2i.reshape(1, 3 * C), W0_u2i,
                             W_post, b_post.reshape(1, 2))
    w_i2u, _, _ = _gru(xt_i, tv_i, Wih_i2u, Whh_i2u,
                       bih_i2u.reshape(1, 3 * C),
                       bhh_i2u.reshape(1, 3 * C), W0_i2u,
                       W_post, b_post.reshape(1, 2))

    h_u2i = _htables(xu, dsi_u2i, w_u2i)
    h_i2u = _htables(xi, dsi_i2u, w_i2u)
    ones = jnp.ones((C,), jnp.float32)
    v_sl = _aggregate(h_u2i, s01, d01, ddi_u2i.reshape(NPAD), ones)
    u_sl = _aggregate(h_i2u, s10, d10, ddi_i2u.reshape(NPAD),
                      wrow.reshape(C))

    lpad = jnp.zeros((ELPAD - EL,), jnp.int32)
    l0 = jnp.concatenate([edge_label_index[0].astype(jnp.int32), lpad])
    l1 = jnp.concatenate([edge_label_index[1].astype(jnp.int32), lpad])
    l0 = l0.reshape(NSUB * NCORE, LCH, C)
    l1 = l1.reshape(NSUB * NCORE, LCH, C)
    partial = _link(l0, l1, u_sl, v_sl, bsum.reshape(16))
    red = _reduce(partial.reshape(ELPAD * 16 // C, C), bsum)
    return jnp.transpose(red, (0, 2, 1)).reshape(ELPAD)[:EL]
